# Initial kernel scaffold; baseline (speedup 1.0000x reference)
import jax
import jax.numpy as jnp
from jax.experimental import pallas as pl


def kernel(x, h, edge_index, edge_attr, W_e1, b_e1, W_e2, b_e2, W_att, b_att, W_x1, b_x1, W_x2, b_x2, W_h1, b_h1, W_h2, b_h2):
    raise NotImplementedError("write your pallas kernel here")


# speedup vs baseline: 4.3899x; 4.3899x over previous
import functools

import jax
import jax.numpy as jnp
from jax import lax
from jax.experimental import pallas as pl
from jax.experimental.pallas import tpu as pltpu
from jax.experimental.pallas import tpu_sc as plsc

N = 10000
D = 128
M = 64
DW = 128
CH = 128
NC = 2
NS = 16
NW = NC * NS
BN = 1000
BE = 4096
NSEG = 10112
RPT = NSEG // NS


def _silu(v):
    return v * jax.nn.sigmoid(v)


def _dg(a, b):
    return lax.dot_general(a, b, (((1,), (1,)), ((), ())),
                           preferred_element_type=jnp.float32)


def _table_body(h_ref, xp_ref, wr_ref, wc_ref, rt_ref, ct_ref):
    hh = h_ref[...]
    xp = xp_ref[...]
    rt_ref[...] = jnp.concatenate([_dg(hh, wr_ref[...]), xp], axis=1)
    ct_ref[...] = jnp.concatenate([_dg(hh, wc_ref[...]), -xp], axis=1)


def _edge_body(u_ref, ea_ref, wrad_ref, wea_ref, be1_ref, we2_ref, be2_ref,
               watt_ref, batt_ref, out_ref):
    u = u_ref[...]
    pq = u[:, :M]
    xd = u[:, M:DW]
    s2b = _dg(xd * xd, jnp.ones((M, M), jnp.float32))
    radb = jnp.sqrt(s2b)
    z = _silu(pq + radb * wrad_ref[...] + _dg(ea_ref[...], wea_ref[...])
              + be1_ref[...])
    m = _silu(_dg(z, we2_ref[...]) + be2_ref[...])
    att = jax.nn.sigmoid(_dg(m, watt_ref[...]) + batt_ref[...])
    msg = m * att
    out_ref[...] = jnp.concatenate([msg, jnp.zeros_like(msg)], axis=1)


def _node_body(h_ref, ms_ref, wh1a_ref, wh1b_ref, bh1_ref, wh2_ref, bh2_ref,
               out_ref):
    hh = h_ref[...]
    ms = ms_ref[0][:, :M] + ms_ref[1][:, :M]
    t = _silu(_dg(hh, wh1a_ref[...]) + _dg(ms, wh1b_ref[...]) + bh1_ref[...])
    out_ref[...] = hh + _dg(t, wh2_ref[...]) + bh2_ref[...]


def _gather_body(K, rt_hbm, ct_hbm, rowg_hbm, colg_hbm, u_hbm,
                 rowv, colv, rbuf0, cbuf0, rbuf1, cbuf1,
                 semr0, semc0, semr1, semc1):
    c = lax.axis_index("c")
    s = lax.axis_index("s")
    wid = c * NS + s
    base = wid * (K * CH)
    pltpu.sync_copy(rowg_hbm.at[wid], rowv)
    pltpu.sync_copy(colg_hbm.at[wid], colv)

    def _add(rb, cb):
        def body(i, _):
            for j in range(DW // 16):
                sl = pl.ds(j * 16, 16)
                rb[i, sl] = rb[i, sl] + cb[i, sl]
            return 0
        lax.fori_loop(0, CH, body, 0, unroll=2)

    def _issue(k, rb, cb, sr, sc_):
        pltpu.async_copy(rt_hbm.at[rowv.at[k]], rb, sr)
        pltpu.async_copy(ct_hbm.at[colv.at[k]], cb, sc_)

    def _wait(k, rb, cb, sr, sc_):
        pltpu.make_async_copy(rt_hbm.at[rowv.at[k]], rb, sr).wait()
        pltpu.make_async_copy(ct_hbm.at[colv.at[k]], cb, sc_).wait()

    def _emit(k, rb):
        pltpu.sync_copy(rb, u_hbm.at[pl.ds(base + k * CH, CH)])

    _issue(0, rbuf0, cbuf0, semr0, semc0)

    def loop_body(k2, _):
        k0 = 2 * k2
        _issue(k0 + 1, rbuf1, cbuf1, semr1, semc1)
        _wait(k0, rbuf0, cbuf0, semr0, semc0)
        _add(rbuf0, cbuf0)
        _emit(k0, rbuf0)
        _issue(k0 + 2, rbuf0, cbuf0, semr0, semc0)
        _wait(k0 + 1, rbuf1, cbuf1, semr1, semc1)
        _add(rbuf1, cbuf1)
        _emit(k0 + 1, rbuf1)
        return 0

    lax.fori_loop(0, (K - 1) // 2, loop_body, 0)
    kl = K - 1
    _wait(kl, rbuf0, cbuf0, semr0, semc0)
    _add(rbuf0, cbuf0)
    _emit(kl, rbuf0)


_RPT_CHUNKS = [(o, min(128, RPT - o)) for o in range(0, RPT, 128)]


def _scatter_body(K, msg_hbm, sidx_hbm, out_hbm, sidxv, mbuf, msum_sh):
    c = lax.axis_index("c")
    s = lax.axis_index("s")
    wid = c * NS + s
    base = wid * (K * CH)
    pltpu.sync_copy(sidx_hbm.at[wid], sidxv)

    def zrow(i, _):
        for j in range(DW // 16):
            mbuf[i, pl.ds(j * 16, 16)] = jnp.zeros((16,), jnp.float32)
        return 0
    lax.fori_loop(0, CH, zrow, 0)
    for off, sz in _RPT_CHUNKS:
        pltpu.sync_copy(mbuf.at[pl.ds(0, sz)],
                        msum_sh.at[pl.ds(s * RPT + off, sz)])
    plsc.subcore_barrier()

    def body(k, _):
        pltpu.sync_copy(msg_hbm.at[pl.ds(base + k * CH, CH)], mbuf)
        pltpu.sync_copy(mbuf, msum_sh.at[sidxv.at[k]], add=True)
        return 0
    lax.fori_loop(0, K, body, 0)
    plsc.subcore_barrier()

    for off, sz in _RPT_CHUNKS:
        pltpu.sync_copy(msum_sh.at[pl.ds(s * RPT + off, sz)],
                        mbuf.at[pl.ds(0, sz)])
        pltpu.sync_copy(mbuf.at[pl.ds(0, sz)],
                        out_hbm.at[c, pl.ds(s * RPT + off, sz)])


def kernel(x, h, edge_index, edge_attr, W_e1, b_e1, W_e2, b_e2, W_att, b_att,
           W_x1, b_x1, W_x2, b_x2, W_h1, b_h1, W_h2, b_h2):
    E = edge_index.shape[1]
    K = -(-E // (NW * CH))
    if K % 2 == 0:
        K += 1
    E_pad = NW * K * CH
    KE = E_pad // BE
    f32 = jnp.float32

    row = edge_index[0]
    col = edge_index[1]
    pad = E_pad - E
    row_g = jnp.concatenate([row, jnp.zeros((pad,), jnp.int32)]).reshape(NW, K, CH)
    col_g = jnp.concatenate([col, jnp.zeros((pad,), jnp.int32)]).reshape(NW, K, CH)
    sidx = jnp.concatenate([row, jnp.full((pad,), N, jnp.int32)]).reshape(NW, K, CH)
    ea8 = jnp.concatenate([edge_attr, jnp.zeros((pad, 4), f32)])
    ea8 = jnp.pad(ea8, ((0, 0), (0, 4)))
    xp = jnp.pad(x, ((0, 0), (0, M - 3)))

    wr = W_e1[:, :D]
    wc = W_e1[:, D:2 * D]
    wrad = W_e1[:, 2 * D:2 * D + 1].reshape(1, M)
    wea = jnp.pad(W_e1[:, 2 * D + 1:], ((0, 0), (0, 4)))
    be1 = b_e1.reshape(1, M)
    be2 = b_e2.reshape(1, M)
    watt_rep = jnp.tile(W_att, (M, 1))
    batt_rep = jnp.tile(b_att.reshape(1, 1), (1, M))
    wh1a = W_h1[:, :D]
    wh1b = W_h1[:, D:]
    bh1 = b_h1.reshape(1, M)
    bh2 = b_h2.reshape(1, D)

    rt, ct = pl.pallas_call(
        _table_body,
        grid=(N // BN,),
        in_specs=[
            pl.BlockSpec((BN, D), lambda i: (i, 0)),
            pl.BlockSpec((BN, M), lambda i: (i, 0)),
            pl.BlockSpec((M, D), lambda i: (0, 0)),
            pl.BlockSpec((M, D), lambda i: (0, 0)),
        ],
        out_specs=[
            pl.BlockSpec((BN, DW), lambda i: (i, 0)),
            pl.BlockSpec((BN, DW), lambda i: (i, 0)),
        ],
        out_shape=[
            jax.ShapeDtypeStruct((N, DW), f32),
            jax.ShapeDtypeStruct((N, DW), f32),
        ],
    )(h, xp, wr, wc)

    mesh = plsc.VectorSubcoreMesh(core_axis_name="c", subcore_axis_name="s")
    u = pl.kernel(
        functools.partial(_gather_body, K),
        out_type=jax.ShapeDtypeStruct((E_pad, DW), f32),
        mesh=mesh,
        scratch_types=[
            pltpu.VMEM((K, CH), jnp.int32),
            pltpu.VMEM((K, CH), jnp.int32),
            pltpu.VMEM((CH, DW), f32),
            pltpu.VMEM((CH, DW), f32),
            pltpu.VMEM((CH, DW), f32),
            pltpu.VMEM((CH, DW), f32),
            pltpu.SemaphoreType.DMA,
            pltpu.SemaphoreType.DMA,
            pltpu.SemaphoreType.DMA,
            pltpu.SemaphoreType.DMA,
        ],
    )(rt, ct, row_g, col_g)

    msg = pl.pallas_call(
        _edge_body,
        grid=(KE,),
        in_specs=[
            pl.BlockSpec((BE, DW), lambda i: (i, 0)),
            pl.BlockSpec((BE, 8), lambda i: (i, 0)),
            pl.BlockSpec((1, M), lambda i: (0, 0)),
            pl.BlockSpec((M, 8), lambda i: (0, 0)),
            pl.BlockSpec((1, M), lambda i: (0, 0)),
            pl.BlockSpec((M, M), lambda i: (0, 0)),
            pl.BlockSpec((1, M), lambda i: (0, 0)),
            pl.BlockSpec((M, M), lambda i: (0, 0)),
            pl.BlockSpec((1, M), lambda i: (0, 0)),
        ],
        out_specs=pl.BlockSpec((BE, DW), lambda i: (i, 0)),
        out_shape=jax.ShapeDtypeStruct((E_pad, DW), f32),
    )(u, ea8, wrad, wea, be1, W_e2, be2, watt_rep, batt_rep)

    msum2 = pl.kernel(
        functools.partial(_scatter_body, K),
        out_type=jax.ShapeDtypeStruct((NC, NSEG, DW), f32),
        mesh=mesh,
        scratch_types=[
            pltpu.VMEM((K, CH), jnp.int32),
            pltpu.VMEM((CH, DW), f32),
            pltpu.VMEM_SHARED((NSEG, DW), f32),
        ],
    )(msg, sidx)

    out = pl.pallas_call(
        _node_body,
        grid=(N // BN,),
        in_specs=[
            pl.BlockSpec((BN, D), lambda i: (i, 0)),
            pl.BlockSpec((NC, BN, DW), lambda i: (0, i, 0)),
            pl.BlockSpec((M, D), lambda i: (0, 0)),
            pl.BlockSpec((M, M), lambda i: (0, 0)),
            pl.BlockSpec((1, M), lambda i: (0, 0)),
            pl.BlockSpec((D, M), lambda i: (0, 0)),
            pl.BlockSpec((1, D), lambda i: (0, 0)),
        ],
        out_specs=pl.BlockSpec((BN, D), lambda i: (i, 0)),
        out_shape=jax.ShapeDtypeStruct((N, D), f32),
    )(h, msum2, wh1a, wh1b, bh1, W_h2, bh2)
    return out
